# baseline (speedup 1.0000x reference)
import functools

import jax
import jax.numpy as jnp
from jax import lax
from jax.experimental import pallas as pl
from jax.experimental.pallas import tpu as pltpu
from jax.experimental.pallas import tpu_sc as plsc

D = 64
KP1 = 21
NC = 2
NS = 16
NW = NC * NS
CH = 64
IDX_CHUNK = 112


def _dot_kernel(cw_hbm, cat_hbm, ctab_hbm, xtab_hbm, pos_hbm, neg_hbm,
                cidx_v, catidx_v, crow_v, catrow_v, pos_v, nout_v, sem,
                *, b_per_w, n_chunks, n_neg):
    wid = lax.axis_index("s") * NC + lax.axis_index("c")
    base = wid * b_per_w
    n_gathers = (CH * KP1) // IDX_CHUNK

    def chunk_body(c, carry):
        cs = base + c * CH
        pltpu.sync_copy(cw_hbm.at[pl.ds(cs, CH)], cidx_v)
        pltpu.sync_copy(cat_hbm.at[pl.ds(cs * KP1, CH * KP1)], catidx_v)
        copies = [pltpu.async_copy(ctab_hbm.at[cidx_v], crow_v, sem)]
        for j in range(n_gathers):
            copies.append(pltpu.async_copy(
                xtab_hbm.at[catidx_v.at[pl.ds(j * IDX_CHUNK, IDX_CHUNK)]],
                catrow_v.at[pl.ds(j * IDX_CHUNK, IDX_CHUNK)],
                sem))
        for cp in copies:
            cp.wait()

        lane = lax.broadcasted_iota(jnp.int32, (16,), 0)
        m15 = lane == 15
        colc = [lane + dd * 16 for dd in range(4)]

        def b_body(b, carry2):
            bsplat = jnp.zeros((16,), jnp.int32) + b
            cregs = [plsc.load_gather(crow_v, [bsplat, colc[dd]])
                     for dd in range(4)]
            for j in range(KP1):
                psplat = bsplat * KP1 + j
                t = cregs[0] * plsc.load_gather(catrow_v, [psplat, colc[0]])
                for dd in range(1, 4):
                    t = t + cregs[dd] * plsc.load_gather(
                        catrow_v, [psplat, colc[dd]])
                s = jnp.cumsum(t)
                if j == 0:
                    plsc.store_scatter(pos_v, [bsplat], s, mask=m15)
                else:
                    plsc.store_scatter(nout_v, [bsplat * n_neg + (j - 1)], -s,
                                       mask=m15)
            return carry2

        lax.fori_loop(0, CH, b_body, 0)
        pltpu.sync_copy(pos_v, pos_hbm.at[pl.ds(cs, CH)])
        pltpu.sync_copy(nout_v, neg_hbm.at[pl.ds(cs * n_neg, CH * n_neg)])
        return carry

    lax.fori_loop(0, n_chunks, chunk_body, 0)


@jax.jit
def kernel(center_words, context_words, neg_samples, center_table, context_table):
    B, K = neg_samples.shape
    b_per_w = B // NW
    n_chunks = b_per_w // CH
    cw = center_words.astype(jnp.int32)
    cat = jnp.concatenate(
        [context_words.astype(jnp.int32)[:, None], neg_samples.astype(jnp.int32)],
        axis=1).reshape(B * KP1)

    mesh = plsc.VectorSubcoreMesh(core_axis_name="c", subcore_axis_name="s")
    run = pl.kernel(
        functools.partial(_dot_kernel, b_per_w=b_per_w, n_chunks=n_chunks,
                          n_neg=K),
        out_type=[
            jax.ShapeDtypeStruct((B,), jnp.float32),
            jax.ShapeDtypeStruct((B * K,), jnp.float32),
        ],
        mesh=mesh,
        compiler_params=pltpu.CompilerParams(needs_layout_passes=False,
                                             use_tc_tiling_on_sc=False),
        scratch_types=[
            pltpu.VMEM((CH,), jnp.int32),
            pltpu.VMEM((CH * KP1,), jnp.int32),
            pltpu.VMEM((CH, D), jnp.float32),
            pltpu.VMEM((CH * KP1, D), jnp.float32),
            pltpu.VMEM((CH,), jnp.float32),
            pltpu.VMEM((CH * K,), jnp.float32),
            pltpu.SemaphoreType.DMA,
        ],
    )
    pos, neg = run(cw, cat, center_table, context_table)
    return pos, neg.reshape(B, K)


# speedup vs baseline: 1.4231x; 1.4231x over previous
import functools

import jax
import jax.numpy as jnp
from jax import lax
from jax.experimental import layout as jlayout
from jax.experimental import pallas as pl
from jax.experimental.pallas import tpu as pltpu
from jax.experimental.pallas import tpu_sc as plsc

D = 64
KP1 = 21
NC = 2
NS = 16
NW = NC * NS
CH = 64
IDX_CHUNK = 112


def _dot_kernel(cw_hbm, cat_hbm, ctab_hbm, xtab_hbm, pos_hbm, neg_hbm,
                cidx_v, catidx_v, crow_v, catrow_v, pos_v, nout_v, sem,
                *, b_per_w, n_chunks, n_neg):
    wid = lax.axis_index("s") * NC + lax.axis_index("c")
    base = wid * b_per_w
    n_gathers = (CH * KP1) // IDX_CHUNK

    def chunk_body(c, carry):
        cs = base + c * CH
        pltpu.sync_copy(cw_hbm.at[pl.ds(cs, CH)], cidx_v)
        pltpu.sync_copy(cat_hbm.at[pl.ds(cs * KP1, CH * KP1)], catidx_v)
        copies = [pltpu.async_copy(ctab_hbm.at[cidx_v], crow_v, sem)]
        for j in range(n_gathers):
            copies.append(pltpu.async_copy(
                xtab_hbm.at[catidx_v.at[pl.ds(j * IDX_CHUNK, IDX_CHUNK)]],
                catrow_v.at[pl.ds(j * IDX_CHUNK, IDX_CHUNK)],
                sem))
        for cp in copies:
            cp.wait()

        lane = lax.broadcasted_iota(jnp.int32, (16,), 0)
        m15 = lane == 15
        colc = [lane + dd * 16 for dd in range(4)]

        def b_body(b, carry2):
            bsplat = jnp.zeros((16,), jnp.int32) + b
            cregs = [plsc.load_gather(crow_v, [bsplat, colc[dd]])
                     for dd in range(4)]
            for j in range(KP1):
                psplat = bsplat * KP1 + j
                t = cregs[0] * plsc.load_gather(catrow_v, [psplat, colc[0]])
                for dd in range(1, 4):
                    t = t + cregs[dd] * plsc.load_gather(
                        catrow_v, [psplat, colc[dd]])
                s = jnp.cumsum(t)
                if j == 0:
                    plsc.store_scatter(pos_v, [bsplat], s, mask=m15)
                else:
                    plsc.store_scatter(nout_v, [bsplat * n_neg + (j - 1)], -s,
                                       mask=m15)
            return carry2

        lax.fori_loop(0, CH, b_body, 0)
        pltpu.sync_copy(pos_v, pos_hbm.at[pl.ds(cs, CH)])
        pltpu.sync_copy(nout_v, neg_hbm.at[pl.ds(cs * n_neg, CH * n_neg)])
        return carry

    lax.fori_loop(0, n_chunks, chunk_body, 0)


@jax.jit
def kernel(center_words, context_words, neg_samples, center_table, context_table):
    B, K = neg_samples.shape
    b_per_w = B // NW
    n_chunks = b_per_w // CH
    cw = center_words.astype(jnp.int32)
    cat = jnp.concatenate(
        [context_words.astype(jnp.int32)[:, None], neg_samples.astype(jnp.int32)],
        axis=1).reshape(B * KP1)

    mesh = plsc.VectorSubcoreMesh(core_axis_name="c", subcore_axis_name="s")
    run = pl.kernel(
        functools.partial(_dot_kernel, b_per_w=b_per_w, n_chunks=n_chunks,
                          n_neg=K),
        out_type=[
            jax.ShapeDtypeStruct((B,), jnp.float32),
            jax.ShapeDtypeStruct((B * K,), jnp.float32),
        ],
        mesh=mesh,
        compiler_params=pltpu.CompilerParams(needs_layout_passes=False,
                                             use_tc_tiling_on_sc=False),
        scratch_types=[
            pltpu.VMEM((CH,), jnp.int32),
            pltpu.VMEM((CH * KP1,), jnp.int32),
            pltpu.VMEM((CH, D), jnp.float32),
            pltpu.VMEM((CH * KP1, D), jnp.float32),
            pltpu.VMEM((CH,), jnp.float32),
            pltpu.VMEM((CH * K,), jnp.float32),
            pltpu.SemaphoreType.DMA,
        ],
    )
    fmt = jlayout.Layout(major_to_minor=(0, 1), tiling=((8,),))
    ctab = jlayout.with_layout_constraint(center_table, fmt)
    xtab = jlayout.with_layout_constraint(context_table, fmt)
    pos, neg = run(cw, cat, ctab, xtab)
    return pos, neg.reshape(B, K)
